# baseline (speedup 1.0000x reference)
import functools

import jax
import jax.numpy as jnp
from jax.experimental import pallas as pl
from jax.experimental.pallas import tpu as pltpu

T = 200
H = 256
D = 512


def _lstm_kernel(imgs_ref, cells_ref, emb_i_ref, emb_c_ref, w_ih_t_ref,
                 w_hh_t_ref, b_ref, fc_wt_ref, fc_b_ref, out_ref,
                 z_ref, hs_ref):
    img_ids = imgs_ref[:]
    cell_ids = cells_ref[:]
    oh_img = (jax.lax.broadcasted_iota(jnp.int32, (T, 900), 1)
              == img_ids).astype(jnp.float32)
    oh_cell = (jax.lax.broadcasted_iota(jnp.int32, (T, 8), 1)
               == cell_ids).astype(jnp.float32)
    x_img = jnp.dot(oh_img, emb_i_ref[:], preferred_element_type=jnp.float32)
    x_cell = jnp.dot(oh_cell, emb_c_ref[:], preferred_element_type=jnp.float32)

    z = (jnp.dot(x_img, w_ih_t_ref[0:H, :], preferred_element_type=jnp.float32)
         + jnp.dot(x_cell, w_ih_t_ref[H:D, :], preferred_element_type=jnp.float32)
         + b_ref[:])
    z_ref[:] = z

    def step(t, carry):
        h, c = carry
        hb = h.astype(jnp.bfloat16)
        g = (z_ref[pl.ds(t, 1), :]
             + jnp.dot(hb[:, 0:128], w_hh_t_ref[0:128, :],
                       preferred_element_type=jnp.float32)
             + jnp.dot(hb[:, 128:256], w_hh_t_ref[128:256, :],
                       preferred_element_type=jnp.float32))
        i = jax.nn.sigmoid(g[:, 0:H])
        f = jax.nn.sigmoid(g[:, H:2 * H])
        gg = jnp.tanh(g[:, 2 * H:3 * H])
        o = jax.nn.sigmoid(g[:, 3 * H:4 * H])
        c_new = f * c + i * gg
        h_new = o * jnp.tanh(c_new)
        hs_ref[pl.ds(t, 1), :] = h_new
        return h_new, c_new

    h0 = jnp.zeros((1, H), jnp.float32)
    c0 = jnp.zeros((1, H), jnp.float32)
    jax.lax.fori_loop(0, T, step, (h0, c0), unroll=4)

    logits = jnp.dot(hs_ref[:], fc_wt_ref[:],
                     preferred_element_type=jnp.float32) + fc_b_ref[:]
    out_ref[:] = jax.nn.sigmoid(logits)


@functools.partial(jax.jit, static_argnames=("interpret",))
def _run(imgs15, cells15, emb_indice, emb_cell, w_ih_t, w_hh_t, b, fc_wt,
         fc_b, interpret=False):
    return pl.pallas_call(
        _lstm_kernel,
        out_shape=jax.ShapeDtypeStruct((T, 2), jnp.float32),
        scratch_shapes=[
            pltpu.VMEM((T, 4 * H), jnp.float32),
            pltpu.VMEM((T, H), jnp.float32),
        ],
        interpret=interpret,
    )(imgs15, cells15, emb_indice, emb_cell, w_ih_t, w_hh_t, b, fc_wt, fc_b)


def kernel(cells, imgs, emb_cell, emb_indice, W_ih, W_hh, b_ih, b_hh, fc_w,
           fc_b):
    imgs15 = imgs[:, -1].astype(jnp.int32).reshape(T, 1)
    cells15 = cells[:, -1].astype(jnp.int32).reshape(T, 1)
    emb_cell8 = jnp.pad(emb_cell, ((0, 3), (0, 0)))
    b = (b_ih + b_hh).reshape(1, 4 * H)
    return _run(imgs15, cells15, emb_indice, emb_cell8, W_ih.T,
                W_hh.T.astype(jnp.bfloat16), b, fc_w.T, fc_b.reshape(1, 2))


# speedup vs baseline: 1.2108x; 1.2108x over previous
import functools

import jax
import jax.numpy as jnp
from jax.experimental import pallas as pl
from jax.experimental.pallas import tpu as pltpu

T = 200
H = 256
D = 512

_DNT = (((1,), (1,)), ((), ()))


def _dot_t(x, w):
    return jax.lax.dot_general(x, w, _DNT, preferred_element_type=jnp.float32)


def _lstm_kernel(imgs_ref, cells_ref, emb_i_ref, emb_c_ref, w_ih_ref,
                 w_hh_t_ref, b_ih_ref, b_hh_ref, fc_w_ref, fc_b_ref, out_ref,
                 z_ref, hs_ref):
    img_ids = imgs_ref[:, 15:16]
    cell_ids = cells_ref[:, 15:16]
    oh_img = (jax.lax.broadcasted_iota(jnp.int32, (T, 900), 1)
              == img_ids).astype(jnp.float32)
    oh_cell = (jax.lax.broadcasted_iota(jnp.int32, (T, 8), 1)
               == cell_ids).astype(jnp.float32)
    x_img = jnp.dot(oh_img, emb_i_ref[:], preferred_element_type=jnp.float32)
    x_cell = jnp.dot(oh_cell, emb_c_ref[:], preferred_element_type=jnp.float32)

    z = (_dot_t(x_img, w_ih_ref[:, 0:H])
         + _dot_t(x_cell, w_ih_ref[:, H:D])
         + b_ih_ref[:] + b_hh_ref[:])
    z_ref[:] = z

    h = jnp.zeros((1, H), jnp.float32)
    c = jnp.zeros((1, H), jnp.float32)
    for t in range(T):
        g = z_ref[t:t + 1, :] + jnp.dot(
            h.astype(jnp.bfloat16), w_hh_t_ref[:],
            preferred_element_type=jnp.float32)
        i = jax.nn.sigmoid(g[:, 0:H])
        f = jax.nn.sigmoid(g[:, H:2 * H])
        gg = jnp.tanh(g[:, 2 * H:3 * H])
        o = jax.nn.sigmoid(g[:, 3 * H:4 * H])
        c = f * c + i * gg
        h = o * jnp.tanh(c)
        hs_ref[t:t + 1, :] = h

    logits = _dot_t(hs_ref[:], fc_w_ref[:]) + fc_b_ref[:]
    out_ref[:] = jax.nn.sigmoid(logits)


@functools.partial(jax.jit, static_argnames=("interpret",))
def _run(imgs, cells, emb_indice, emb_cell, w_ih, w_hh_t, b_ih, b_hh, fc_w,
         fc_b, interpret=False):
    return pl.pallas_call(
        _lstm_kernel,
        out_shape=jax.ShapeDtypeStruct((T, 2), jnp.float32),
        scratch_shapes=[
            pltpu.VMEM((T, 4 * H), jnp.float32),
            pltpu.VMEM((T, H), jnp.float32),
        ],
        interpret=interpret,
    )(imgs, cells, emb_indice, emb_cell, w_ih, w_hh_t, b_ih, b_hh, fc_w,
      fc_b)


def kernel(cells, imgs, emb_cell, emb_indice, W_ih, W_hh, b_ih, b_hh, fc_w,
           fc_b):
    emb_cell8 = jnp.pad(emb_cell, ((0, 3), (0, 0)))
    return _run(imgs.astype(jnp.int32), cells.astype(jnp.int32), emb_indice,
                emb_cell8, W_ih, W_hh.T.astype(jnp.bfloat16),
                b_ih.reshape(1, 4 * H), b_hh.reshape(1, 4 * H), fc_w,
                fc_b.reshape(1, 2))


# speedup vs baseline: 1.2194x; 1.0071x over previous
import functools

import jax
import jax.numpy as jnp
from jax.experimental import pallas as pl
from jax.experimental.pallas import tpu as pltpu

T = 200
H = 256
D = 512

_DNT = (((1,), (1,)), ((), ()))


def _dot_t(x, w):
    return jax.lax.dot_general(x, w, _DNT, preferred_element_type=jnp.float32)


def _lstm_kernel(imgs_ref, cells_ref, emb_i_ref, emb_c_ref, w_ih_ref,
                 w_hh_ref, b_ih_ref, b_hh_ref, fc_w_ref, fc_b_ref, out_ref,
                 z_ref, hs_ref, w_hh_t_ref):
    w_hh_t_ref[:] = w_hh_ref[:].T

    img_ids = imgs_ref[:, 15:16]
    cell_ids = cells_ref[:, 15:16]
    oh_img = (jax.lax.broadcasted_iota(jnp.int32, (T, 900), 1)
              == img_ids).astype(jnp.float32)
    oh_cell = (jax.lax.broadcasted_iota(jnp.int32, (T, 8), 1)
               == cell_ids).astype(jnp.float32)
    x_img = jnp.dot(oh_img, emb_i_ref[:], preferred_element_type=jnp.float32)
    x_cell = jnp.dot(oh_cell, emb_c_ref[:], preferred_element_type=jnp.float32)

    z = (_dot_t(x_img, w_ih_ref[:, 0:H])
         + _dot_t(x_cell, w_ih_ref[:, H:D])
         + b_ih_ref[:] + b_hh_ref[:])
    z_ref[:] = z

    h = jnp.zeros((1, H), jnp.float32)
    c = jnp.zeros((1, H), jnp.float32)
    for t in range(T):
        g = z_ref[t:t + 1, :] + jnp.dot(
            h.astype(jnp.bfloat16), w_hh_t_ref[:],
            preferred_element_type=jnp.float32)
        i = jax.nn.sigmoid(g[:, 0:H])
        f = jax.nn.sigmoid(g[:, H:2 * H])
        gg = jnp.tanh(g[:, 2 * H:3 * H])
        o = jax.nn.sigmoid(g[:, 3 * H:4 * H])
        c = f * c + i * gg
        h = o * jnp.tanh(c)
        hs_ref[t:t + 1, :] = h

    logits = _dot_t(hs_ref[:], fc_w_ref[:]) + fc_b_ref[:]
    out_ref[:] = jax.nn.sigmoid(logits)


@functools.partial(jax.jit, static_argnames=("interpret",))
def _run(imgs, cells, emb_indice, emb_cell, w_ih, w_hh_t, b_ih, b_hh, fc_w,
         fc_b, interpret=False):
    return pl.pallas_call(
        _lstm_kernel,
        out_shape=jax.ShapeDtypeStruct((T, 2), jnp.float32),
        scratch_shapes=[
            pltpu.VMEM((T, 4 * H), jnp.float32),
            pltpu.VMEM((T, H), jnp.float32),
            pltpu.VMEM((H, 4 * H), jnp.bfloat16),
        ],
        interpret=interpret,
    )(imgs, cells, emb_indice, emb_cell, w_ih, w_hh_t, b_ih, b_hh, fc_w,
      fc_b)


def kernel(cells, imgs, emb_cell, emb_indice, W_ih, W_hh, b_ih, b_hh, fc_w,
           fc_b):
    emb_cell8 = jnp.pad(emb_cell, ((0, 3), (0, 0)))
    return _run(imgs.astype(jnp.int32), cells.astype(jnp.int32), emb_indice,
                emb_cell8, W_ih, W_hh.astype(jnp.bfloat16),
                b_ih.reshape(1, 4 * H), b_hh.reshape(1, 4 * H), fc_w,
                fc_b.reshape(1, 2))


# speedup vs baseline: 1.2725x; 1.0435x over previous
import functools

import jax
import jax.numpy as jnp
from jax.experimental import pallas as pl
from jax.experimental.pallas import tpu as pltpu

T = 200
H = 256
D = 512

_DNT = (((1,), (1,)), ((), ()))


def _dot_t(x, w):
    return jax.lax.dot_general(x, w, _DNT, preferred_element_type=jnp.float32)


def _lstm_kernel(imgs_ref, cells_ref, emb_i_ref, emb_c_ref, w_ih_ref,
                 w_hh_ref, b_ih_ref, b_hh_ref, fc_w_ref, fc_b_ref, out_ref,
                 z_ref, hs_ref, w_hh_t_ref):
    w_hh_t_ref[:] = w_hh_ref[:].astype(jnp.bfloat16).T

    img_ids = imgs_ref[:, 15:16]
    cell_ids = cells_ref[:, 15:16]
    oh_img = (jax.lax.broadcasted_iota(jnp.int32, (T, 900), 1)
              == img_ids).astype(jnp.float32)
    oh_cell = (jax.lax.broadcasted_iota(jnp.int32, (T, 8), 1)
               == cell_ids).astype(jnp.float32)
    x_img = jnp.dot(oh_img, emb_i_ref[:], preferred_element_type=jnp.float32)
    x_cell = jnp.dot(oh_cell, emb_c_ref[:], preferred_element_type=jnp.float32)

    z = (_dot_t(x_img, w_ih_ref[:, 0:H])
         + _dot_t(x_cell, w_ih_ref[:, H:D])
         + b_ih_ref[:] + b_hh_ref[:])
    z_ref[:] = z

    h = jnp.zeros((1, H), jnp.float32)
    c = jnp.zeros((1, H), jnp.float32)
    for t in range(T):
        g = z_ref[t:t + 1, :] + jnp.dot(
            h.astype(jnp.bfloat16), w_hh_t_ref[:],
            preferred_element_type=jnp.float32)
        i = jax.nn.sigmoid(g[:, 0:H])
        f = jax.nn.sigmoid(g[:, H:2 * H])
        gg = jnp.tanh(g[:, 2 * H:3 * H])
        o = jax.nn.sigmoid(g[:, 3 * H:4 * H])
        c = f * c + i * gg
        h = o * jnp.tanh(c)
        hs_ref[t:t + 1, :] = h

    logits = _dot_t(hs_ref[:], fc_w_ref[:]) + fc_b_ref[:]
    out_ref[:] = jax.nn.sigmoid(logits)


@functools.partial(jax.jit, static_argnames=("interpret",))
def _run(imgs, cells, emb_indice, emb_cell, w_ih, w_hh_t, b_ih, b_hh, fc_w,
         fc_b, interpret=False):
    return pl.pallas_call(
        _lstm_kernel,
        out_shape=jax.ShapeDtypeStruct((T, 2), jnp.float32),
        scratch_shapes=[
            pltpu.VMEM((T, 4 * H), jnp.float32),
            pltpu.VMEM((T, H), jnp.float32),
            pltpu.VMEM((H, 4 * H), jnp.bfloat16),
        ],
        interpret=interpret,
    )(imgs, cells, emb_indice, emb_cell, w_ih, w_hh_t, b_ih, b_hh, fc_w,
      fc_b)


def kernel(cells, imgs, emb_cell, emb_indice, W_ih, W_hh, b_ih, b_hh, fc_w,
           fc_b):
    emb_cell8 = jnp.pad(emb_cell, ((0, 3), (0, 0)))
    return _run(imgs.astype(jnp.int32), cells.astype(jnp.int32), emb_indice,
                emb_cell8, W_ih, W_hh,
                b_ih.reshape(1, 4 * H), b_hh.reshape(1, 4 * H), fc_w,
                fc_b.reshape(1, 2))


# speedup vs baseline: 1.3264x; 1.0424x over previous
import functools

import jax
import jax.numpy as jnp
from jax.experimental import pallas as pl
from jax.experimental.pallas import tpu as pltpu

T = 200
H = 256
D = 512

_DNT = (((1,), (1,)), ((), ()))


def _dot_t(x, w):
    return jax.lax.dot_general(x, w, _DNT, preferred_element_type=jnp.float32)


def _lstm_kernel(imgs_ref, cells_ref, emb_i_ref, emb_c_ref, w_ih_ref,
                 w_hh_ref, b_ih_ref, b_hh_ref, fc_w_ref, fc_b_ref, out_ref,
                 z_ref, hs_ref, w_hh_t_ref):
    w_hh_t_ref[:] = w_hh_ref[:].astype(jnp.bfloat16).T

    img_ids = imgs_ref[:, 15:16]
    cell_ids = cells_ref[:, 15:16]
    oh_img = (jax.lax.broadcasted_iota(jnp.int32, (T, 900), 1)
              == img_ids).astype(jnp.float32)
    oh_cell = (jax.lax.broadcasted_iota(jnp.int32, (T, 8), 1)
               == cell_ids).astype(jnp.float32)
    x_img = jnp.dot(oh_img, emb_i_ref[:], preferred_element_type=jnp.float32)
    x_cell = jnp.dot(oh_cell, emb_c_ref[:], preferred_element_type=jnp.float32)

    z = (_dot_t(x_img, w_ih_ref[:, 0:H])
         + _dot_t(x_cell, w_ih_ref[:, H:D])
         + b_ih_ref[:] + b_hh_ref[:])
    z_ref[:] = z.astype(jnp.bfloat16)

    h = jnp.zeros((1, H), jnp.float32)
    c = jnp.zeros((1, H), jnp.float32)
    for t in range(T):
        g = z_ref[t:t + 1, :] + jnp.dot(
            h.astype(jnp.bfloat16), w_hh_t_ref[:],
            preferred_element_type=jnp.float32)
        i = jax.nn.sigmoid(g[:, 0:H])
        f = jax.nn.sigmoid(g[:, H:2 * H])
        gg = jnp.tanh(g[:, 2 * H:3 * H])
        o = jax.nn.sigmoid(g[:, 3 * H:4 * H])
        c = f * c + i * gg
        h = o * jnp.tanh(c)
        hs_ref[t:t + 1, :] = h

    logits = _dot_t(hs_ref[:], fc_w_ref[:]) + fc_b_ref[:]
    out_ref[:] = jax.nn.sigmoid(logits)


@functools.partial(jax.jit, static_argnames=("interpret",))
def _run(imgs, cells, emb_indice, emb_cell, w_ih, w_hh_t, b_ih, b_hh, fc_w,
         fc_b, interpret=False):
    return pl.pallas_call(
        _lstm_kernel,
        out_shape=jax.ShapeDtypeStruct((T, 2), jnp.float32),
        scratch_shapes=[
            pltpu.VMEM((T, 4 * H), jnp.bfloat16),
            pltpu.VMEM((T, H), jnp.float32),
            pltpu.VMEM((H, 4 * H), jnp.bfloat16),
        ],
        interpret=interpret,
    )(imgs, cells, emb_indice, emb_cell, w_ih, w_hh_t, b_ih, b_hh, fc_w,
      fc_b)


def kernel(cells, imgs, emb_cell, emb_indice, W_ih, W_hh, b_ih, b_hh, fc_w,
           fc_b):
    emb_cell8 = jnp.pad(emb_cell, ((0, 3), (0, 0)))
    return _run(imgs.astype(jnp.int32), cells.astype(jnp.int32), emb_indice,
                emb_cell8, W_ih, W_hh,
                b_ih.reshape(1, 4 * H), b_hh.reshape(1, 4 * H), fc_w,
                fc_b.reshape(1, 2))
